# Initial kernel scaffold; baseline (speedup 1.0000x reference)
import jax
import jax.numpy as jnp
from jax.experimental import pallas as pl


def kernel(x, x_in, emb_table, enc_Wih, enc_Whh, enc_bih, enc_bhh, fc_mu_W, fc_mu_b, fc_logvar_W, fc_logvar_b, dec_Wih, dec_Whh, dec_bih, dec_bhh, fc_out_W, fc_out_b, fc_z_W, fc_z_b):
    raise NotImplementedError("write your pallas kernel here")


# speedup vs baseline: 1.0250x; 1.0250x over previous
import functools

import jax
import jax.numpy as jnp
from jax import lax
from jax.experimental import pallas as pl
from jax.experimental.pallas import tpu as pltpu
from jax.experimental.pallas import tpu_sc as plsc

B, S, V, E, H, Z = 1024, 50, 1000, 64, 128, 32
H3 = 3 * H

_PREC = jax.lax.Precision.HIGHEST


def _f32(shape):
    return jax.ShapeDtypeStruct(shape, jnp.float32)


def _sigmoid(x):
    return 1.0 / (1.0 + jnp.exp(-x))


def _sc_gather_pair(table, idx_a, idx_b):
    ntok = idx_a.shape[0]
    info = plsc.get_sparse_core_info()
    nw = info.num_cores * info.num_subcores
    bpw = ntok // nw
    assert ntok % (8 * nw) == 0 and bpw % 8 == 0
    chunk = 128
    nfull, rem = divmod(bpw, chunk)
    mesh = plsc.VectorSubcoreMesh(core_axis_name="c", subcore_axis_name="s")

    @functools.partial(
        pl.kernel,
        out_type=[_f32((ntok, E)), _f32((ntok, E))],
        mesh=mesh,
        compiler_params=pltpu.CompilerParams(use_tc_tiling_on_sc=False),
        scratch_types=[
            pltpu.VMEM((bpw,), jnp.int32),
            pltpu.VMEM((bpw, E), jnp.float32),
            pltpu.SemaphoreType.DMA,
        ],
    )
    def gather_kernel(table_hbm, ia_hbm, ib_hbm, oa_hbm, ob_hbm,
                      idx_v, rows_v, sem):
        wid = lax.axis_index("s") * info.num_cores + lax.axis_index("c")
        base = pl.multiple_of(wid * bpw, 8)

        def one(idx_hbm, out_hbm):
            pltpu.sync_copy(idx_hbm.at[pl.ds(base, bpw)], idx_v)
            copies = []
            for c in range(nfull):
                copies.append(pltpu.async_copy(
                    table_hbm.at[idx_v.at[pl.ds(c * chunk, chunk)]],
                    rows_v.at[pl.ds(c * chunk, chunk)], sem))
            if rem:
                copies.append(pltpu.async_copy(
                    table_hbm.at[idx_v.at[pl.ds(nfull * chunk, rem)]],
                    rows_v.at[pl.ds(nfull * chunk, rem)], sem))
            for cp in copies:
                cp.wait()
            pltpu.sync_copy(rows_v, out_hbm.at[pl.ds(base, bpw)])

        one(ia_hbm, oa_hbm)
        one(ib_hbm, ob_hbm)

    return gather_kernel(table, idx_a, idx_b)


def _gru_step(e_t, h, wih_t, whh_t, bih, bhh):
    gi = jnp.dot(e_t, wih_t, precision=_PREC,
                 preferred_element_type=jnp.float32) + bih
    gh = jnp.dot(h, whh_t, precision=_PREC,
                 preferred_element_type=jnp.float32) + bhh
    r = _sigmoid(gi[:, :H] + gh[:, :H])
    zg = _sigmoid(gi[:, H:2 * H] + gh[:, H:2 * H])
    n = jnp.tanh(gi[:, 2 * H:] + r * gh[:, 2 * H:])
    return (1.0 - zg) * n + zg * h


def _encoder_body(emb_ref, wih_ref, whh_ref, bih_ref, bhh_ref,
                  muw_ref, mub_ref, lvw_ref, lvb_ref, zw_ref, zb_ref,
                  eps_ref, mu_ref, lv_ref, h0_ref):
    wih_t = wih_ref[...]
    whh_t = whh_ref[...]
    bih = bih_ref[...]
    bhh = bhh_ref[...]
    def step(t, h):
        return _gru_step(emb_ref[t], h, wih_t, whh_t, bih, bhh)

    h = lax.fori_loop(0, S, step, jnp.zeros((B, H), jnp.float32))
    mu = jnp.dot(h, muw_ref[...], precision=_PREC,
                 preferred_element_type=jnp.float32) + mub_ref[...]
    lv = jnp.dot(h, lvw_ref[...], precision=_PREC,
                 preferred_element_type=jnp.float32) + lvb_ref[...]
    z = mu + eps_ref[...] * jnp.exp(0.5 * lv)
    h0 = jnp.dot(z, zw_ref[...], precision=_PREC,
                 preferred_element_type=jnp.float32) + zb_ref[...]
    mu_ref[...] = mu
    lv_ref[...] = lv
    h0_ref[...] = h0


def _decoder_body(emb_ref, h0_ref, wih_ref, whh_ref, bih_ref, bhh_ref,
                  outs_ref):
    wih_t = wih_ref[...]
    whh_t = whh_ref[...]
    bih = bih_ref[...]
    bhh = bhh_ref[...]
    def step(t, h):
        h_new = _gru_step(emb_ref[t], h, wih_t, whh_t, bih, bhh)
        outs_ref[t] = h_new
        return h_new

    lax.fori_loop(0, S, step, h0_ref[...])


def _logits_body(outs_ref, w_ref, b_ref, out_ref):
    bb = out_ref.shape[0]
    hs = jnp.swapaxes(outs_ref[...], 0, 1)
    flat = hs.reshape(bb * S, H)
    logits = jnp.dot(flat, w_ref[...], precision=_PREC,
                     preferred_element_type=jnp.float32) + b_ref[...]
    out_ref[...] = logits.reshape(bb, S, V)


def kernel(x, x_in, emb_table, enc_Wih, enc_Whh, enc_bih, enc_bhh,
           fc_mu_W, fc_mu_b, fc_logvar_W, fc_logvar_b,
           dec_Wih, dec_Whh, dec_bih, dec_bhh,
           fc_out_W, fc_out_b, fc_z_W, fc_z_b):
    idx_x = jnp.swapaxes(x, 0, 1).reshape(-1).astype(jnp.int32)
    idx_in = jnp.swapaxes(x_in, 0, 1).reshape(-1).astype(jnp.int32)
    eps = jax.random.normal(jax.random.key(42), (B, Z), dtype=jnp.float32)

    enc_wih_t = enc_Wih.T
    enc_whh_t = enc_Whh.T
    enc_bih2 = enc_bih.reshape(1, H3)
    enc_bhh2 = enc_bhh.reshape(1, H3)
    dec_wih_t = dec_Wih.T
    dec_whh_t = dec_Whh.T
    dec_bih2 = dec_bih.reshape(1, H3)
    dec_bhh2 = dec_bhh.reshape(1, H3)
    muw_t = fc_mu_W.T
    lvw_t = fc_logvar_W.T
    mub2 = fc_mu_b.reshape(1, Z)
    lvb2 = fc_logvar_b.reshape(1, Z)
    zw_t = fc_z_W.T
    zb2 = fc_z_b.reshape(1, H)
    outw_t = fc_out_W.T
    outb2 = fc_out_b.reshape(1, V)

    emb_x_flat, emb_in_flat = _sc_gather_pair(emb_table, idx_x, idx_in)
    emb_x = emb_x_flat.reshape(S, B, E)
    emb_in = emb_in_flat.reshape(S, B, E)

    mu, logvar, h0 = pl.pallas_call(
        _encoder_body,
        out_shape=[_f32((B, Z)), _f32((B, Z)), _f32((B, H))],
    )(emb_x, enc_wih_t, enc_whh_t, enc_bih2, enc_bhh2,
      muw_t, mub2, lvw_t, lvb2, zw_t, zb2, eps)

    outs = pl.pallas_call(
        _decoder_body,
        out_shape=_f32((S, B, H)),
    )(emb_in, h0, dec_wih_t, dec_whh_t, dec_bih2, dec_bhh2)

    bb = 64
    nb = B // bb
    logits = pl.pallas_call(
        _logits_body,
        grid=(nb,),
        in_specs=[
            pl.BlockSpec((S, bb, H), lambda i: (0, i, 0)),
            pl.BlockSpec((H, V), lambda i: (0, 0)),
            pl.BlockSpec((1, V), lambda i: (0, 0)),
        ],
        out_specs=pl.BlockSpec((bb, S, V), lambda i: (i, 0, 0)),
        out_shape=_f32((B, S, V)),
    )(outs, outw_t, outb2)

    return (logits, mu, logvar)


# speedup vs baseline: 1.7146x; 1.6728x over previous
import functools

import jax
import jax.numpy as jnp
from jax import lax
from jax.experimental import pallas as pl
from jax.experimental.pallas import tpu as pltpu
from jax.experimental.pallas import tpu_sc as plsc

B, S, V, E, H, Z = 1024, 50, 1000, 64, 128, 32
H3 = 3 * H

_PREC = jax.lax.Precision.DEFAULT


def _f32(shape):
    return jax.ShapeDtypeStruct(shape, jnp.float32)


def _sigmoid(x):
    return 0.5 * jnp.tanh(0.5 * x) + 0.5


def _sc_gather(table, idx):
    ntok = idx.shape[0]
    info = plsc.get_sparse_core_info()
    nw = info.num_cores * info.num_subcores
    bpw = ntok // nw
    assert ntok % (8 * nw) == 0 and bpw % 8 == 0
    chunk = 128
    nfull, rem = divmod(bpw, chunk)
    mesh = plsc.VectorSubcoreMesh(core_axis_name="c", subcore_axis_name="s")

    @functools.partial(
        pl.kernel,
        out_type=_f32((ntok, E)),
        mesh=mesh,
        compiler_params=pltpu.CompilerParams(use_tc_tiling_on_sc=False),
        scratch_types=[
            pltpu.VMEM((bpw,), jnp.int32),
            pltpu.VMEM((bpw, E), jnp.float32),
            pltpu.SemaphoreType.DMA,
        ],
    )
    def gather_kernel(table_hbm, idx_hbm, out_hbm, idx_v, rows_v, sem):
        wid = lax.axis_index("s") * info.num_cores + lax.axis_index("c")
        base = pl.multiple_of(wid * bpw, 8)
        pltpu.sync_copy(idx_hbm.at[pl.ds(base, bpw)], idx_v)
        copies = []
        for c in range(nfull):
            copies.append(pltpu.async_copy(
                table_hbm.at[idx_v.at[pl.ds(c * chunk, chunk)]],
                rows_v.at[pl.ds(c * chunk, chunk)], sem))
        if rem:
            copies.append(pltpu.async_copy(
                table_hbm.at[idx_v.at[pl.ds(nfull * chunk, rem)]],
                rows_v.at[pl.ds(nfull * chunk, rem)], sem))
        for cp in copies:
            cp.wait()
        pltpu.sync_copy(rows_v, out_hbm.at[pl.ds(base, bpw)])

    return gather_kernel(table, idx)


def _gru_step(e_t, h, wih_t, whh_t, bih, bhh):
    gi = jnp.dot(e_t, wih_t, precision=_PREC,
                 preferred_element_type=jnp.float32) + bih
    gh = jnp.dot(h, whh_t, precision=_PREC,
                 preferred_element_type=jnp.float32) + bhh
    r = _sigmoid(gi[:, :H] + gh[:, :H])
    zg = _sigmoid(gi[:, H:2 * H] + gh[:, H:2 * H])
    n = jnp.tanh(gi[:, 2 * H:] + r * gh[:, 2 * H:])
    return (1.0 - zg) * n + zg * h


def _encoder_body(emb_ref, wih_ref, whh_ref, bih_ref, bhh_ref,
                  muw_ref, mub_ref, lvw_ref, lvb_ref, zw_ref, zb_ref,
                  eps_ref, mu_ref, lv_ref, h0_ref):
    wih_t = wih_ref[...]
    whh_t = whh_ref[...]
    bih = bih_ref[...]
    bhh = bhh_ref[...]
    def step(t, h):
        return _gru_step(emb_ref[t], h, wih_t, whh_t, bih, bhh)

    h = lax.fori_loop(0, S, step, jnp.zeros((B, H), jnp.float32))
    mu = jnp.dot(h, muw_ref[...], precision=_PREC,
                 preferred_element_type=jnp.float32) + mub_ref[...]
    lv = jnp.dot(h, lvw_ref[...], precision=_PREC,
                 preferred_element_type=jnp.float32) + lvb_ref[...]
    z = mu + eps_ref[...] * jnp.exp(0.5 * lv)
    h0 = jnp.dot(z, zw_ref[...], precision=_PREC,
                 preferred_element_type=jnp.float32) + zb_ref[...]
    mu_ref[...] = mu
    lv_ref[...] = lv
    h0_ref[...] = h0


def _decoder_body(emb_ref, h0_ref, wih_ref, whh_ref, bih_ref, bhh_ref,
                  outs_ref):
    wih_t = wih_ref[...]
    whh_t = whh_ref[...]
    bih = bih_ref[...]
    bhh = bhh_ref[...]
    def step(t, h):
        h_new = _gru_step(emb_ref[t], h, wih_t, whh_t, bih, bhh)
        outs_ref[t] = h_new
        return h_new

    lax.fori_loop(0, S, step, h0_ref[...])


def _logits_body(outs_ref, w_ref, b_ref, out_ref):
    bb = out_ref.shape[0]
    hs = jnp.swapaxes(outs_ref[...], 0, 1)
    flat = hs.reshape(bb * S, H)
    logits = jnp.dot(flat, w_ref[...], precision=_PREC,
                     preferred_element_type=jnp.float32) + b_ref[...]
    out_ref[...] = logits.reshape(bb, S, V)


def kernel(x, x_in, emb_table, enc_Wih, enc_Whh, enc_bih, enc_bhh,
           fc_mu_W, fc_mu_b, fc_logvar_W, fc_logvar_b,
           dec_Wih, dec_Whh, dec_bih, dec_bhh,
           fc_out_W, fc_out_b, fc_z_W, fc_z_b):
    idx_x = jnp.swapaxes(x, 0, 1).reshape(-1).astype(jnp.int32)
    idx_in = jnp.swapaxes(x_in, 0, 1).reshape(-1).astype(jnp.int32)
    eps = jax.random.normal(jax.random.key(42), (B, Z), dtype=jnp.float32)

    enc_wih_t = enc_Wih.T
    enc_whh_t = enc_Whh.T
    enc_bih2 = enc_bih.reshape(1, H3)
    enc_bhh2 = enc_bhh.reshape(1, H3)
    dec_wih_t = dec_Wih.T
    dec_whh_t = dec_Whh.T
    dec_bih2 = dec_bih.reshape(1, H3)
    dec_bhh2 = dec_bhh.reshape(1, H3)
    muw_t = fc_mu_W.T
    lvw_t = fc_logvar_W.T
    mub2 = fc_mu_b.reshape(1, Z)
    lvb2 = fc_logvar_b.reshape(1, Z)
    zw_t = fc_z_W.T
    zb2 = fc_z_b.reshape(1, H)
    outw_t = fc_out_W.T
    outb2 = fc_out_b.reshape(1, V)

    emb_x = _sc_gather(emb_table, idx_x).reshape(S, B, E)
    emb_in = _sc_gather(emb_table, idx_in).reshape(S, B, E)

    mu, logvar, h0 = pl.pallas_call(
        _encoder_body,
        out_shape=[_f32((B, Z)), _f32((B, Z)), _f32((B, H))],
    )(emb_x, enc_wih_t, enc_whh_t, enc_bih2, enc_bhh2,
      muw_t, mub2, lvw_t, lvb2, zw_t, zb2, eps)

    outs = pl.pallas_call(
        _decoder_body,
        out_shape=_f32((S, B, H)),
    )(emb_in, h0, dec_wih_t, dec_whh_t, dec_bih2, dec_bhh2)

    bb = 64
    nb = B // bb
    logits = pl.pallas_call(
        _logits_body,
        grid=(nb,),
        in_specs=[
            pl.BlockSpec((S, bb, H), lambda i: (0, i, 0)),
            pl.BlockSpec((H, V), lambda i: (0, 0)),
            pl.BlockSpec((1, V), lambda i: (0, 0)),
        ],
        out_specs=pl.BlockSpec((bb, S, V), lambda i: (i, 0, 0)),
        out_shape=_f32((B, S, V)),
    )(outs, outw_t, outb2)

    return (logits, mu, logvar)


# speedup vs baseline: 3.0295x; 1.7668x over previous
import functools

import jax
import jax.numpy as jnp
from jax import lax
from jax.experimental import pallas as pl
from jax.experimental.pallas import tpu as pltpu
from jax.experimental.pallas import tpu_sc as plsc

B, S, V, E, H, Z = 1024, 50, 1000, 64, 128, 32
H3 = 3 * H

_PREC = jax.lax.Precision.DEFAULT


def _f32(shape):
    return jax.ShapeDtypeStruct(shape, jnp.float32)


def _sigmoid(x):
    return 0.5 * jnp.tanh(0.5 * x) + 0.5


def _sc_gather(table, idx):
    ntok = idx.shape[0]
    info = plsc.get_sparse_core_info()
    nw = info.num_cores * info.num_subcores
    bpw = ntok // nw
    assert ntok % (8 * nw) == 0 and bpw % 8 == 0
    chunk = 128
    nfull, rem = divmod(bpw, chunk)
    mesh = plsc.VectorSubcoreMesh(core_axis_name="c", subcore_axis_name="s")

    @functools.partial(
        pl.kernel,
        out_type=_f32((ntok, E)),
        mesh=mesh,
        compiler_params=pltpu.CompilerParams(use_tc_tiling_on_sc=False),
        scratch_types=[
            pltpu.VMEM((bpw,), jnp.int32),
            pltpu.VMEM((bpw, E), jnp.float32),
            pltpu.SemaphoreType.DMA,
        ],
    )
    def gather_kernel(table_hbm, idx_hbm, out_hbm, idx_v, rows_v, sem):
        wid = lax.axis_index("s") * info.num_cores + lax.axis_index("c")
        base = pl.multiple_of(wid * bpw, 8)
        pltpu.sync_copy(idx_hbm.at[pl.ds(base, bpw)], idx_v)
        copies = []
        for c in range(nfull):
            copies.append(pltpu.async_copy(
                table_hbm.at[idx_v.at[pl.ds(c * chunk, chunk)]],
                rows_v.at[pl.ds(c * chunk, chunk)], sem))
        if rem:
            copies.append(pltpu.async_copy(
                table_hbm.at[idx_v.at[pl.ds(nfull * chunk, rem)]],
                rows_v.at[pl.ds(nfull * chunk, rem)], sem))
        for cp in copies:
            cp.wait()
        pltpu.sync_copy(rows_v, out_hbm.at[pl.ds(base, bpw)])

    return gather_kernel(table, idx)


def _gru_step(e_t, h, wih_t, whh_t, bih, bhh):
    gi = jnp.dot(e_t, wih_t, precision=_PREC,
                 preferred_element_type=jnp.float32) + bih
    gh = jnp.dot(h, whh_t, precision=_PREC,
                 preferred_element_type=jnp.float32) + bhh
    r = _sigmoid(gi[:, :H] + gh[:, :H])
    zg = _sigmoid(gi[:, H:2 * H] + gh[:, H:2 * H])
    n = jnp.tanh(gi[:, 2 * H:] + r * gh[:, 2 * H:])
    return (1.0 - zg) * n + zg * h


def _encoder_body(emb_ref, wih_ref, whh_ref, bih_ref, bhh_ref,
                  muw_ref, mub_ref, lvw_ref, lvb_ref, zw_ref, zb_ref,
                  eps_ref, mu_ref, lv_ref, h0_ref):
    wih_t = wih_ref[...]
    whh_t = whh_ref[...]
    bih = bih_ref[...]
    bhh = bhh_ref[...]
    def step(t, h):
        return _gru_step(emb_ref[t], h, wih_t, whh_t, bih, bhh)

    h = lax.fori_loop(0, S, step, jnp.zeros((B, H), jnp.float32))
    mu = jnp.dot(h, muw_ref[...], precision=_PREC,
                 preferred_element_type=jnp.float32) + mub_ref[...]
    lv = jnp.dot(h, lvw_ref[...], precision=_PREC,
                 preferred_element_type=jnp.float32) + lvb_ref[...]
    z = mu + eps_ref[...] * jnp.exp(0.5 * lv)
    h0 = jnp.dot(z, zw_ref[...], precision=_PREC,
                 preferred_element_type=jnp.float32) + zb_ref[...]
    mu_ref[...] = mu
    lv_ref[...] = lv
    h0_ref[...] = h0


def _decoder_body(emb_ref, h0_ref, wih_ref, whh_ref, bih_ref, bhh_ref,
                  outs_ref):
    wih_t = wih_ref[...]
    whh_t = whh_ref[...]
    bih = bih_ref[...]
    bhh = bhh_ref[...]
    def step(t, h):
        h_new = _gru_step(emb_ref[t], h, wih_t, whh_t, bih, bhh)
        outs_ref[t] = h_new
        return h_new

    lax.fori_loop(0, S, step, h0_ref[...])


def _logits_body(outs_ref, w_ref, b_ref, out_ref):
    h_t = jnp.swapaxes(outs_ref[0], 0, 1)
    acc = jnp.dot(w_ref[...], h_t, precision=_PREC,
                  preferred_element_type=jnp.float32)
    out_ref[0] = acc + b_ref[...]


def kernel(x, x_in, emb_table, enc_Wih, enc_Whh, enc_bih, enc_bhh,
           fc_mu_W, fc_mu_b, fc_logvar_W, fc_logvar_b,
           dec_Wih, dec_Whh, dec_bih, dec_bhh,
           fc_out_W, fc_out_b, fc_z_W, fc_z_b):
    idx_x = jnp.swapaxes(x, 0, 1).reshape(-1).astype(jnp.int32)
    idx_in = jnp.swapaxes(x_in, 0, 1).reshape(-1).astype(jnp.int32)
    eps = jax.random.normal(jax.random.key(42), (B, Z), dtype=jnp.float32)

    enc_wih_t = enc_Wih.T
    enc_whh_t = enc_Whh.T
    enc_bih2 = enc_bih.reshape(1, H3)
    enc_bhh2 = enc_bhh.reshape(1, H3)
    dec_wih_t = dec_Wih.T
    dec_whh_t = dec_Whh.T
    dec_bih2 = dec_bih.reshape(1, H3)
    dec_bhh2 = dec_bhh.reshape(1, H3)
    muw_t = fc_mu_W.T
    lvw_t = fc_logvar_W.T
    mub2 = fc_mu_b.reshape(1, Z)
    lvb2 = fc_logvar_b.reshape(1, Z)
    zw_t = fc_z_W.T
    zb2 = fc_z_b.reshape(1, H)
    outb2 = fc_out_b.reshape(V, 1)

    emb_x = _sc_gather(emb_table, idx_x).reshape(S, B, E)
    emb_in = _sc_gather(emb_table, idx_in).reshape(S, B, E)

    mu, logvar, h0 = pl.pallas_call(
        _encoder_body,
        out_shape=[_f32((B, Z)), _f32((B, Z)), _f32((B, H))],
    )(emb_x, enc_wih_t, enc_whh_t, enc_bih2, enc_bhh2,
      muw_t, mub2, lvw_t, lvb2, zw_t, zb2, eps)

    outs = pl.pallas_call(
        _decoder_body,
        out_shape=_f32((S, B, H)),
    )(emb_in, h0, dec_wih_t, dec_whh_t, dec_bih2, dec_bhh2)

    logits_t = pl.pallas_call(
        _logits_body,
        grid=(S,),
        in_specs=[
            pl.BlockSpec((1, B, H), lambda s: (s, 0, 0)),
            pl.BlockSpec((V, H), lambda s: (0, 0)),
            pl.BlockSpec((V, 1), lambda s: (0, 0)),
        ],
        out_specs=pl.BlockSpec((1, V, B), lambda s: (s, 0, 0)),
        out_shape=_f32((S, V, B)),
    )(outs, fc_out_W, outb2)
    logits = jnp.transpose(logits_t, (2, 0, 1))

    return (logits, mu, logvar)
